# baseline (device time: 215929 ns/iter reference)
import jax
import jax.numpy as jnp
from jax import lax
from jax.experimental import pallas as pl
from jax.experimental.pallas import tpu as pltpu

N_DEV = 16
B, SQ, D = 4, 256, 1024
H_PER = 8
DH = 128
ROWS = B * SQ
CH = ROWS // N_DEV
SCALE = 0.08838834764831843

N_STEPS = N_DEV - 1
STAGE = 2 * N_STEPS


def _allreduce(partial):

    def body(p_ref, out_ref, comm_ref, send_sems, recv_sems):
        my = lax.axis_index("i")
        left = (my - 1) % N_DEV
        right = (my + 1) % N_DEV

        barrier_sem = pltpu.get_barrier_semaphore()
        for nbr in (left, right):
            pl.semaphore_signal(
                barrier_sem, inc=1,
                device_id=(nbr,), device_id_type=pl.DeviceIdType.MESH,
            )
        pl.semaphore_wait(barrier_sem, 2)

        comm_ref[STAGE, :, :] = p_ref[pl.ds(my * CH, CH), :]

        for s in range(N_STEPS):
            src_slot = STAGE if s == 0 else s - 1
            rdma = pltpu.make_async_remote_copy(
                src_ref=comm_ref.at[src_slot],
                dst_ref=comm_ref.at[s],
                send_sem=send_sems.at[s],
                recv_sem=recv_sems.at[s],
                device_id=(right,),
                device_id_type=pl.DeviceIdType.MESH,
            )
            rdma.start()
            rdma.wait()
            rc = (my - 1 - s) % N_DEV
            comm_ref[s, :, :] = comm_ref[s, :, :] + p_ref[pl.ds(rc * CH, CH), :]

        r = (my + 1) % N_DEV
        out_ref[pl.ds(r * CH, CH), :] = comm_ref[N_STEPS - 1, :, :]

        for s in range(N_STEPS):
            src_slot = (N_STEPS - 1) if s == 0 else N_STEPS + s - 1
            rdma = pltpu.make_async_remote_copy(
                src_ref=comm_ref.at[src_slot],
                dst_ref=comm_ref.at[N_STEPS + s],
                send_sem=send_sems.at[N_STEPS + s],
                recv_sem=recv_sems.at[N_STEPS + s],
                device_id=(right,),
                device_id_type=pl.DeviceIdType.MESH,
            )
            rdma.start()
            rdma.wait()
            rc = (my - s) % N_DEV
            out_ref[pl.ds(rc * CH, CH), :] = comm_ref[N_STEPS + s, :, :]

    return pl.pallas_call(
        body,
        out_shape=jax.ShapeDtypeStruct((ROWS, D), jnp.float32),
        in_specs=[pl.BlockSpec(memory_space=pltpu.VMEM)],
        out_specs=pl.BlockSpec(memory_space=pltpu.VMEM),
        scratch_shapes=[
            pltpu.VMEM((2 * N_STEPS + 1, CH, D), jnp.float32),
            pltpu.SemaphoreType.DMA((2 * N_STEPS,)),
            pltpu.SemaphoreType.DMA((2 * N_STEPS,)),
        ],
        compiler_params=pltpu.CompilerParams(collective_id=0),
    )(partial)


def kernel(x, Wq, Wo, K_ext, V_ext):
    my = lax.axis_index("i")

    Q = (x.reshape(B * SQ, D) @ Wq).reshape(B, SQ, H_PER, DH)

    h0 = my * H_PER
    K = lax.dynamic_slice_in_dim(K_ext, h0, H_PER, axis=2)
    V = lax.dynamic_slice_in_dim(V_ext, h0, H_PER, axis=2)

    s = jnp.einsum("bihd,bjhd->bhij", Q, K) * SCALE
    p = jax.nn.softmax(s, axis=-1)
    attn = jnp.einsum("bhij,bjhd->bihd", p, V)

    partial = attn.reshape(B * SQ, H_PER * DH) @ Wo

    return _allreduce(partial).reshape(B, SQ, D)


# device time: 151565 ns/iter; 1.4247x vs baseline; 1.4247x over previous
import jax
import jax.numpy as jnp
from jax import lax
from jax.experimental import pallas as pl
from jax.experimental.pallas import tpu as pltpu

N_DEV = 16
B, SQ, D = 4, 256, 1024
H_PER = 8
DH = 128
ROWS = B * SQ
SCALE = 0.08838834764831843

QR = ROWS // 4
HALF = D // 2
ZC = 2 * QR // 4
BH = HALF // 2


def _allreduce(partial):

    def body(p_ref, out_ref, stage_a, abuf_p, abuf_m, zbuf, bstage,
             bbuf_p, bbuf_m, cbuf_p, cbuf_m, send_sems, recv_sems):
        my = lax.axis_index("i")
        z = my // 4
        q = my % 4
        p_right = z * 4 + (q + 1) % 4
        p_left = z * 4 + (q - 1) % 4
        z_right = ((z + 1) % 4) * 4 + q
        z_left = ((z - 1) % 4) * 4 + q

        barrier_sem = pltpu.get_barrier_semaphore()
        for nbr in (p_right, p_left, z_right, z_left):
            pl.semaphore_signal(
                barrier_sem, inc=1,
                device_id=(nbr,), device_id_type=pl.DeviceIdType.MESH,
            )
        pl.semaphore_wait(barrier_sem, 4)

        def rdma_start(src, dst, sem_idx, dev):
            r = pltpu.make_async_remote_copy(
                src_ref=src, dst_ref=dst,
                send_sem=send_sems.at[sem_idx],
                recv_sem=recv_sems.at[sem_idx],
                device_id=(dev,), device_id_type=pl.DeviceIdType.MESH,
            )
            r.start()
            return r

        stage_a[0, :, :] = p_ref[pl.ds(q * QR, QR), 0:HALF]
        stage_a[1, :, :] = p_ref[pl.ds(((q + 1) % 4) * QR, QR), HALF:D]
        for s in range(3):
            src_p = stage_a.at[0] if s == 0 else abuf_p.at[s - 1]
            src_m = stage_a.at[1] if s == 0 else abuf_m.at[s - 1]
            r1 = rdma_start(src_p, abuf_p.at[s], 0 + s, p_right)
            r2 = rdma_start(src_m, abuf_m.at[s], 3 + s, p_left)
            r1.wait()
            r2.wait()
            rc_p = (q - 1 - s) % 4
            rc_m = (q + 2 + s) % 4
            abuf_p[s, :, :] = abuf_p[s, :, :] + p_ref[pl.ds(rc_p * QR, QR), 0:HALF]
            abuf_m[s, :, :] = abuf_m[s, :, :] + p_ref[pl.ds(rc_m * QR, QR), HALF:D]

        zbuf[0:QR, :] = abuf_p[2, :, :]
        zbuf[QR:2 * QR, :] = abuf_m[2, :, :]

        bstage[0, :, :] = zbuf[pl.ds(z * ZC, ZC), 0:BH]
        bstage[1, :, :] = zbuf[pl.ds(((z + 1) % 4) * ZC, ZC), BH:HALF]
        for s in range(3):
            src_p = bstage.at[0] if s == 0 else bbuf_p.at[s - 1]
            src_m = bstage.at[1] if s == 0 else bbuf_m.at[s - 1]
            r1 = rdma_start(src_p, bbuf_p.at[s], 6 + s, z_right)
            r2 = rdma_start(src_m, bbuf_m.at[s], 9 + s, z_left)
            r1.wait()
            r2.wait()
            rc_p = (z - 1 - s) % 4
            rc_m = (z + 2 + s) % 4
            bbuf_p[s, :, :] = bbuf_p[s, :, :] + zbuf[pl.ds(rc_p * ZC, ZC), 0:BH]
            bbuf_m[s, :, :] = bbuf_m[s, :, :] + zbuf[pl.ds(rc_m * ZC, ZC), BH:HALF]
        zbuf[pl.ds(((z + 1) % 4) * ZC, ZC), 0:BH] = bbuf_p[2, :, :]
        zbuf[pl.ds(z * ZC, ZC), BH:HALF] = bbuf_m[2, :, :]
        for s in range(3):
            src_p = bbuf_p.at[2] if s == 0 else bbuf_p.at[2 + s]
            src_m = bbuf_m.at[2] if s == 0 else bbuf_m.at[2 + s]
            r1 = rdma_start(src_p, bbuf_p.at[3 + s], 12 + s, z_right)
            r2 = rdma_start(src_m, bbuf_m.at[3 + s], 15 + s, z_left)
            r1.wait()
            r2.wait()
            zbuf[pl.ds(((z - s) % 4) * ZC, ZC), 0:BH] = bbuf_p[3 + s, :, :]
            zbuf[pl.ds(((z + 1 + s) % 4) * ZC, ZC), BH:HALF] = bbuf_m[3 + s, :, :]

        out_ref[pl.ds(((q + 1) % 4) * QR, QR), 0:HALF] = zbuf[0:QR, :]
        out_ref[pl.ds(q * QR, QR), HALF:D] = zbuf[QR:2 * QR, :]

        for s in range(3):
            src_p = zbuf.at[pl.ds(0, QR)] if s == 0 else cbuf_p.at[s - 1]
            src_m = zbuf.at[pl.ds(QR, QR)] if s == 0 else cbuf_m.at[s - 1]
            r1 = rdma_start(src_p, cbuf_p.at[s], 18 + s, p_right)
            r2 = rdma_start(src_m, cbuf_m.at[s], 21 + s, p_left)
            r1.wait()
            r2.wait()
            out_ref[pl.ds(((q - s) % 4) * QR, QR), 0:HALF] = cbuf_p[s, :, :]
            out_ref[pl.ds(((q + 1 + s) % 4) * QR, QR), HALF:D] = cbuf_m[s, :, :]

    return pl.pallas_call(
        body,
        out_shape=jax.ShapeDtypeStruct((ROWS, D), jnp.float32),
        in_specs=[pl.BlockSpec(memory_space=pltpu.VMEM)],
        out_specs=pl.BlockSpec(memory_space=pltpu.VMEM),
        scratch_shapes=[
            pltpu.VMEM((2, QR, HALF), jnp.float32),
            pltpu.VMEM((3, QR, HALF), jnp.float32),
            pltpu.VMEM((3, QR, HALF), jnp.float32),
            pltpu.VMEM((2 * QR, HALF), jnp.float32),
            pltpu.VMEM((2, ZC, BH), jnp.float32),
            pltpu.VMEM((6, ZC, BH), jnp.float32),
            pltpu.VMEM((6, ZC, BH), jnp.float32),
            pltpu.VMEM((3, QR, HALF), jnp.float32),
            pltpu.VMEM((3, QR, HALF), jnp.float32),
            pltpu.SemaphoreType.DMA((24,)),
            pltpu.SemaphoreType.DMA((24,)),
        ],
        compiler_params=pltpu.CompilerParams(collective_id=0),
    )(partial)


def kernel(x, Wq, Wo, K_ext, V_ext):
    my = lax.axis_index("i")

    Q = (x.reshape(B * SQ, D) @ Wq).reshape(B, SQ, H_PER, DH)

    h0 = my * H_PER
    K = lax.dynamic_slice_in_dim(K_ext, h0, H_PER, axis=2)
    V = lax.dynamic_slice_in_dim(V_ext, h0, H_PER, axis=2)

    s = jnp.einsum("bihd,bjhd->bhij", Q, K) * SCALE
    p = jax.nn.softmax(s, axis=-1)
    attn = jnp.einsum("bhij,bjhd->bihd", p, V)

    partial = attn.reshape(B * SQ, H_PER * DH) @ Wo

    return _allreduce(partial).reshape(B, SQ, D)
